# baseline (speedup 1.0000x reference)
import functools

import jax
import jax.numpy as jnp
from jax import lax
from jax.experimental import pallas as pl
from jax.experimental.pallas import tpu as pltpu
from jax.experimental.pallas import tpu_sc as plsc

D = 768
B = 4
S = 2048
TOKENS = B * S
EPS = 1e-5

SH = S // 2
TOK_H = B * SH

NC, NS = 2, 16
NW = NC * NS
PER_W = TOK_H // NW
W_PER_B = SH // PER_W
CH = 64
NCH = PER_W // CH
NBUF = 2


@functools.lru_cache(maxsize=1)
def _make_gather_rows():
    mesh = plsc.VectorSubcoreMesh(
        core_axis_name="c", subcore_axis_name="s", num_cores=NC, num_subcores=NS
    )

    @functools.partial(
        pl.kernel,
        mesh=mesh,
        out_type=jax.ShapeDtypeStruct((TOK_H, D), jnp.float32),
        scratch_types=[
            pltpu.VMEM((NCH, CH), jnp.int32),
            *[pltpu.VMEM((CH, D), jnp.float32) for _ in range(NBUF)],
            *[pltpu.SemaphoreType.DMA for _ in range(2 * NBUF)],
        ],
    )
    def _gather_rows(ids_hbm, word_hbm, out_hbm, idx_v, *bufs_sems):
        bufs = bufs_sems[:NBUF]
        gsems = bufs_sems[NBUF : 2 * NBUF]
        ssems = bufs_sems[2 * NBUF :]
        wid = lax.axis_index("s") * NC + lax.axis_index("c")
        b = wid // W_PER_B
        s0 = (wid % W_PER_B) * PER_W
        base = wid * PER_W
        for i in range(NCH):
            pltpu.sync_copy(ids_hbm.at[b, pl.ds(s0 + i * CH, CH)], idx_v.at[i])

        def gather(i):
            return pltpu.async_copy(
                word_hbm.at[idx_v.at[i]], bufs[i % NBUF], gsems[i % NBUF]
            )

        def store(i):
            return pltpu.async_copy(
                bufs[i % NBUF], out_hbm.at[pl.ds(base + i * CH, CH)], ssems[i % NBUF]
            )

        gathers = [gather(i) for i in range(NCH)]
        stores = [None] * NCH
        for i in range(NCH):
            gathers[i].wait()
            stores[i] = store(i)
        for i in range(NCH):
            stores[i].wait()

    return _gather_rows


ROWS_BLK = 512
SB_H = SH // ROWS_BLK


def _ln_body(g_ref, pos_ref, tt_ref, type_ref, gamma_ref, beta_ref, _prev_ref, o_ref):
    x = g_ref[...] + pos_ref[...]
    tt = tt_ref[0, :, 0:1].astype(jnp.float32)
    t0 = type_ref[0:1, :]
    t1 = type_ref[1:2, :]
    x = x + t0 + tt * (t1 - t0)
    mu = jnp.mean(x, axis=1, keepdims=True)
    xc = x - mu
    var = jnp.mean(xc * xc, axis=1, keepdims=True)
    y = xc * lax.rsqrt(var + EPS)
    o_ref[0] = y * gamma_ref[...] + beta_ref[...]


def _make_ln_call(half, alias):
    return pl.pallas_call(
        _ln_body,
        grid=(SB_H, B),
        in_specs=[
            pl.BlockSpec((ROWS_BLK, D), lambda sb, b: (b * SB_H + sb, 0)),
            pl.BlockSpec((ROWS_BLK, D), lambda sb, b: (half * SB_H + sb, 0)),
            pl.BlockSpec((1, ROWS_BLK, 1), lambda sb, b: (b, half * SB_H + sb, 0)),
            pl.BlockSpec((2, D), lambda sb, b: (0, 0)),
            pl.BlockSpec((1, D), lambda sb, b: (0, 0)),
            pl.BlockSpec((1, D), lambda sb, b: (0, 0)),
            pl.BlockSpec(memory_space=pl.ANY),
        ],
        out_specs=pl.BlockSpec(
            (1, ROWS_BLK, D), lambda sb, b: (b, half * SB_H + sb, 0)
        ),
        out_shape=jax.ShapeDtypeStruct((B, S, D), jnp.float32),
        input_output_aliases={6: 0} if alias else {},
    )


_ln_calls = (_make_ln_call(0, False), _make_ln_call(1, True))


def kernel(input_ids, token_type_ids, word_emb, pos_emb, type_emb, gamma, beta):
    ids = input_ids.astype(jnp.int32)
    tt = token_type_ids.astype(jnp.int32)[:, :, None]
    gamma2 = gamma.reshape(1, D)
    beta2 = beta.reshape(1, D)
    sc = _make_gather_rows()
    g0 = sc(ids[:, :SH], word_emb)
    g1 = sc(ids[:, SH:], word_emb)
    dummy = jnp.zeros((1,), jnp.float32)
    out = _ln_calls[0](g0, pos_emb, tt, type_emb, gamma2, beta2, dummy)
    out = _ln_calls[1](g1, pos_emb, tt, type_emb, gamma2, beta2, out)
    return out


# speedup vs baseline: 1.0261x; 1.0261x over previous
import functools

import jax
import jax.numpy as jnp
from jax import lax
from jax.experimental import pallas as pl
from jax.experimental.pallas import tpu as pltpu
from jax.experimental.pallas import tpu_sc as plsc

D = 768
B = 4
S = 2048
TOKENS = B * S
EPS = 1e-5

SH = S // 2
TOK_H = B * SH

NC, NS = 2, 16
NW = NC * NS
PER_W = TOK_H // NW
W_PER_B = SH // PER_W
CH = 64
NCH = PER_W // CH
NBUF = 2


@functools.lru_cache(maxsize=1)
def _make_gather_rows():
    mesh = plsc.VectorSubcoreMesh(
        core_axis_name="c", subcore_axis_name="s", num_cores=NC, num_subcores=NS
    )

    @functools.partial(
        pl.kernel,
        mesh=mesh,
        out_type=jax.ShapeDtypeStruct((TOK_H, D), jnp.float32),
        scratch_types=[
            pltpu.VMEM((NCH, CH), jnp.int32),
            *[pltpu.VMEM((CH, D), jnp.float32) for _ in range(NBUF)],
            *[pltpu.SemaphoreType.DMA for _ in range(2 * NBUF)],
        ],
    )
    def _gather_rows(ids_hbm, word_hbm, out_hbm, idx_v, *bufs_sems):
        bufs = bufs_sems[:NBUF]
        gsems = bufs_sems[NBUF : 2 * NBUF]
        ssems = bufs_sems[2 * NBUF :]
        wid = lax.axis_index("s") * NC + lax.axis_index("c")
        b = wid // W_PER_B
        s0 = (wid % W_PER_B) * PER_W
        base = wid * PER_W
        for i in range(NCH):
            pltpu.sync_copy(ids_hbm.at[b, pl.ds(s0 + i * CH, CH)], idx_v.at[i])

        def gather(i):
            return pltpu.async_copy(
                word_hbm.at[idx_v.at[i]], bufs[i % NBUF], gsems[i % NBUF]
            )

        def store(i):
            return pltpu.async_copy(
                bufs[i % NBUF], out_hbm.at[pl.ds(base + i * CH, CH)], ssems[i % NBUF]
            )

        gathers = [gather(i) for i in range(NCH)]
        stores = [None] * NCH
        for i in range(NCH):
            gathers[i].wait()
            stores[i] = store(i)
        for i in range(NCH):
            stores[i].wait()

    return _gather_rows


ROWS_BLK = 1024
SB_H = SH // ROWS_BLK


def _ln_body(g_ref, pos_ref, tt_ref, type_ref, gamma_ref, beta_ref, _prev_ref, o_ref):
    x = g_ref[...] + pos_ref[...]
    tt = tt_ref[0, :, 0:1].astype(jnp.float32)
    t0 = type_ref[0:1, :]
    t1 = type_ref[1:2, :]
    x = x + t0 + tt * (t1 - t0)
    mu = jnp.mean(x, axis=1, keepdims=True)
    xc = x - mu
    var = jnp.mean(xc * xc, axis=1, keepdims=True)
    y = xc * lax.rsqrt(var + EPS)
    o_ref[0] = y * gamma_ref[...] + beta_ref[...]


def _make_ln_call(half, alias):
    return pl.pallas_call(
        _ln_body,
        grid=(SB_H, B),
        in_specs=[
            pl.BlockSpec((ROWS_BLK, D), lambda sb, b: (b * SB_H + sb, 0)),
            pl.BlockSpec((ROWS_BLK, D), lambda sb, b: (half * SB_H + sb, 0)),
            pl.BlockSpec((1, ROWS_BLK, 1), lambda sb, b: (b, half * SB_H + sb, 0)),
            pl.BlockSpec((2, D), lambda sb, b: (0, 0)),
            pl.BlockSpec((1, D), lambda sb, b: (0, 0)),
            pl.BlockSpec((1, D), lambda sb, b: (0, 0)),
            pl.BlockSpec(memory_space=pl.ANY),
        ],
        out_specs=pl.BlockSpec(
            (1, ROWS_BLK, D), lambda sb, b: (b, half * SB_H + sb, 0)
        ),
        out_shape=jax.ShapeDtypeStruct((B, S, D), jnp.float32),
        input_output_aliases={6: 0} if alias else {},
    )


_ln_calls = (_make_ln_call(0, False), _make_ln_call(1, True))


def kernel(input_ids, token_type_ids, word_emb, pos_emb, type_emb, gamma, beta):
    ids = input_ids.astype(jnp.int32)
    tt = token_type_ids.astype(jnp.int32)[:, :, None]
    gamma2 = gamma.reshape(1, D)
    beta2 = beta.reshape(1, D)
    sc = _make_gather_rows()
    g0 = sc(ids[:, :SH], word_emb)
    g1 = sc(ids[:, SH:], word_emb)
    dummy = jnp.zeros((1,), jnp.float32)
    out = _ln_calls[0](g0, pos_emb, tt, type_emb, gamma2, beta2, dummy)
    out = _ln_calls[1](g1, pos_emb, tt, type_emb, gamma2, beta2, out)
    return out
